# baseline (speedup 1.0000x reference)
import functools
import jax, jax.numpy as jnp
from jax import lax
from jax.experimental import pallas as pl
from jax.experimental.pallas import tpu as pltpu
from jax.experimental.pallas import tpu_sc as plsc

F3 = (96, 128)
_mesh = plsc.VectorSubcoreMesh(core_axis_name="c", subcore_axis_name="s")

@functools.partial(
    pl.kernel, mesh=_mesh,
    out_type=jax.ShapeDtypeStruct((3200,) + F3, jnp.float32),
    scratch_types=[pltpu.SemaphoreType.DMA],
)
def _copy(src_hbm, out_hbm, sem):
    wid = lax.axis_index("s") * 2 + lax.axis_index("c")
    i0 = wid * 2
    cps = []
    for di in range(2):
        i = i0 + di
        cps.append(pltpu.async_copy(src_hbm.at[pl.ds(i, 50)],
                                    out_hbm.at[pl.ds(i * 50, 50)], sem))
    for cp in cps:
        cp.wait()

def kernel(x, buffer):
    src = jnp.concatenate([buffer[1:], x], axis=0).reshape((113,) + F3)
    return _copy(src).reshape(64, 50, 64, 64, 3)


# speedup vs baseline: 5.3293x; 5.3293x over previous
import functools

import jax
import jax.numpy as jnp
from jax import lax
from jax.experimental import pallas as pl
from jax.experimental.pallas import tpu as pltpu
from jax.experimental.pallas import tpu_sc as plsc

H, W, C = 64, 64, 3
F = H * W * C
B = 64
BUF = 50
SRC = BUF - 1 + B
NW = 32
I_PER_W = B // NW
WIN = BUF + 1
CH = 3
NC = WIN // CH
NB = 3

_mesh = plsc.VectorSubcoreMesh(core_axis_name="c", subcore_axis_name="s")


@functools.partial(
    pl.kernel,
    mesh=_mesh,
    out_type=jax.ShapeDtypeStruct((B * BUF, F // 128, 128), jnp.float32),
    scratch_types=[
        pltpu.VMEM((NB, CH, F // 128, 128), jnp.float32),
        pltpu.SemaphoreType.DMA,
    ]
    + [pltpu.SemaphoreType.DMA for _ in range(NB)],
)
def _fifo_copy(src_hbm, out_hbm, vbuf, sem_rd, *sem_wr):
    wid = lax.axis_index("s") * 2 + lax.axis_index("c")
    i0 = wid * I_PER_W
    t0 = i0 * BUF
    t1 = (i0 + 1) * BUF

    def start_read(c):
        return pltpu.async_copy(
            src_hbm.at[pl.ds(i0 + c * CH, CH)], vbuf.at[c % NB], sem_rd)

    def start_writes(c):
        b = c % NB
        cps = []
        n0 = min(CH, BUF - c * CH)
        if n0 > 0:
            cps.append(pltpu.async_copy(
                vbuf.at[b].at[pl.ds(0, n0)],
                out_hbm.at[pl.ds(t0 + c * CH, n0)], sem_wr[b]))
        v1 = 1 if c == 0 else 0
        n1 = min(CH - v1, BUF - (c * CH - 1 + v1))
        if n1 > 0:
            cps.append(pltpu.async_copy(
                vbuf.at[b].at[pl.ds(v1, n1)],
                out_hbm.at[pl.ds(t1 + c * CH - 1 + v1, n1)], sem_wr[b]))
        return cps

    pending_wr = [None] * NC
    reads = [None] * NC
    for c in range(min(NB, NC)):
        reads[c] = start_read(c)
    for c in range(NC):
        reads[c].wait()
        pending_wr[c] = start_writes(c)
        nxt = c + NB
        if nxt < NC:
            for cp in pending_wr[c]:
                cp.wait()
            reads[nxt] = start_read(nxt)
    for c in range(max(0, NC - NB), NC):
        if pending_wr[c] is not None:
            for cp in pending_wr[c]:
                cp.wait()


def kernel(x, buffer):
    src = jnp.concatenate([buffer[1:], x], axis=0).reshape(SRC, F // 128, 128)
    out = _fifo_copy(src)
    return out.reshape(B, BUF, H, W, C)
